# Initial kernel scaffold; baseline (speedup 1.0000x reference)
import jax
import jax.numpy as jnp
from jax.experimental import pallas as pl


def kernel(x, token_table, pos_table):
    raise NotImplementedError("write your pallas kernel here")


# speedup vs baseline: 1.6617x; 1.6617x over previous
import functools

import jax
import jax.numpy as jnp
from jax import lax
from jax.experimental import pallas as pl
from jax.experimental.pallas import tpu as pltpu
from jax.experimental.pallas import tpu_sc as plsc

D = 32
CH = 1024
NC = 2
NS = 16
NW = NC * NS


@functools.partial(jax.jit, static_argnums=(0,))
def _lookup_add(B, idx_flat, token_table, pos_table):
    b_per_w = B // NW
    n_chunks = b_per_w // CH
    mesh = plsc.VectorSubcoreMesh(core_axis_name="c", subcore_axis_name="s")

    @functools.partial(
        pl.kernel,
        out_type=jax.ShapeDtypeStruct((B, D), jnp.float32),
        mesh=mesh,
        compiler_params=pltpu.CompilerParams(use_tc_tiling_on_sc=False),
        scratch_types=[
            pltpu.VMEM((CH,), jnp.int32),
            pltpu.VMEM((CH, D), jnp.float32),
            pltpu.VMEM((CH, D), jnp.float32),
            pltpu.SemaphoreType.DMA,
            pltpu.SemaphoreType.DMA,
        ],
    )
    def k(idx_hbm, tok_hbm, pos_hbm, out_hbm, idx_v, buf_a, buf_b, sem_a, sem_b):
        wid = lax.axis_index("s") * NC + lax.axis_index("c")
        w_base = wid * b_per_w

        def chunk_body(c, carry):
            base = pl.multiple_of(w_base + c * CH, CH)
            pltpu.sync_copy(idx_hbm.at[pl.ds(base, CH)], idx_v)
            cp_a = pltpu.async_copy(tok_hbm.at[idx_v], buf_a, sem_a)
            cp_b = pltpu.async_copy(pos_hbm.at[idx_v], buf_b, sem_b)
            cp_a.wait()
            cp_b.wait()

            def add_body(i, acc):
                for j in (0, 16):
                    buf_a[i, pl.ds(j, 16)] = (
                        buf_a[i, pl.ds(j, 16)] + buf_b[i, pl.ds(j, 16)]
                    )
                return acc

            lax.fori_loop(0, CH, add_body, 0)
            pltpu.sync_copy(buf_a, out_hbm.at[pl.ds(base, CH)])
            return carry

        lax.fori_loop(0, n_chunks, chunk_body, 0)

    return k(idx_flat, token_table, pos_table)


def kernel(x, token_table, pos_table):
    batch, seq = x.shape
    B = batch * seq
    out = _lookup_add(B, x.reshape(B), token_table, pos_table)
    return out.reshape(batch, seq, D)


# speedup vs baseline: 1.7764x; 1.0691x over previous
import functools

import jax
import jax.numpy as jnp
from jax import lax
from jax.experimental import pallas as pl
from jax.experimental.pallas import tpu as pltpu
from jax.experimental.pallas import tpu_sc as plsc

D = 32
CH = 1024
NC = 2
NS = 16
NW = NC * NS


@functools.partial(jax.jit, static_argnums=(0,))
def _lookup_add(B, idx_flat, token_table, pos_table):
    b_per_w = B // NW
    n_chunks = b_per_w // CH
    mesh = plsc.VectorSubcoreMesh(core_axis_name="c", subcore_axis_name="s")

    @functools.partial(
        pl.kernel,
        out_type=jax.ShapeDtypeStruct((B, D), jnp.float32),
        mesh=mesh,
        compiler_params=pltpu.CompilerParams(use_tc_tiling_on_sc=False),
        scratch_types=[
            pltpu.VMEM((CH,), jnp.int32),
            pltpu.VMEM((CH, D), jnp.float32),
            pltpu.VMEM((CH, D), jnp.float32),
            pltpu.SemaphoreType.DMA,
            pltpu.SemaphoreType.DMA,
        ],
    )
    def k(idx_hbm, tok_hbm, pos_hbm, out_hbm, idx_v, buf_a, buf_b, sem_a, sem_b):
        wid = lax.axis_index("s") * NC + lax.axis_index("c")
        w_base = wid * b_per_w

        def chunk_body(c, carry):
            base = pl.multiple_of(w_base + c * CH, CH)
            pltpu.sync_copy(idx_hbm.at[pl.ds(base, CH)], idx_v)
            pltpu.async_copy(tok_hbm.at[idx_v], buf_a, sem_a).wait()
            pltpu.async_copy(pos_hbm.at[idx_v], buf_a, sem_b, add=True).wait()
            pltpu.sync_copy(buf_a, out_hbm.at[pl.ds(base, CH)])
            return carry

        lax.fori_loop(0, n_chunks, chunk_body, 0)

    return k(idx_flat, token_table, pos_table)


def kernel(x, token_table, pos_table):
    batch, seq = x.shape
    B = batch * seq
    out = _lookup_add(B, x.reshape(B), token_table, pos_table)
    return out.reshape(batch, seq, D)


# speedup vs baseline: 1.8249x; 1.0273x over previous
import functools

import jax
import jax.numpy as jnp
from jax import lax
from jax.experimental import pallas as pl
from jax.experimental.pallas import tpu as pltpu
from jax.experimental.pallas import tpu_sc as plsc

D = 32
CH = 3200
NC = 2
NS = 16
NW = NC * NS


@functools.partial(jax.jit, static_argnums=(0,))
def _lookup_add(B, idx_flat, token_table, pos_table):
    b_per_w = B // NW
    n_chunks = b_per_w // CH
    mesh = plsc.VectorSubcoreMesh(core_axis_name="c", subcore_axis_name="s")

    @functools.partial(
        pl.kernel,
        out_type=jax.ShapeDtypeStruct((B, D), jnp.float32),
        mesh=mesh,
        compiler_params=pltpu.CompilerParams(use_tc_tiling_on_sc=False),
        scratch_types=[
            pltpu.VMEM((CH,), jnp.int32),
            pltpu.VMEM((CH, D), jnp.float32),
            pltpu.SemaphoreType.DMA,
            pltpu.SemaphoreType.DMA,
        ],
    )
    def k(idx_hbm, tok_hbm, pos_hbm, out_hbm, idx_v, buf_a, sem_a, sem_b):
        wid = lax.axis_index("s") * NC + lax.axis_index("c")
        w_base = wid * b_per_w

        def chunk_body(c, carry):
            base = pl.multiple_of(w_base + c * CH, CH)
            pltpu.sync_copy(idx_hbm.at[pl.ds(base, CH)], idx_v)
            pltpu.async_copy(tok_hbm.at[idx_v], buf_a, sem_a).wait()
            pltpu.async_copy(pos_hbm.at[idx_v], buf_a, sem_b, add=True).wait()
            pltpu.sync_copy(buf_a, out_hbm.at[pl.ds(base, CH)])
            return carry

        lax.fori_loop(0, n_chunks, chunk_body, 0)

    return k(idx_flat, token_table, pos_table)


def kernel(x, token_table, pos_table):
    batch, seq = x.shape
    B = batch * seq
    out = _lookup_add(B, x.reshape(B), token_table, pos_table)
    return out.reshape(batch, seq, D)
